# Initial kernel scaffold; baseline (speedup 1.0000x reference)
import jax
import jax.numpy as jnp
from jax.experimental import pallas as pl


def kernel(x, edge_attr, edge_index, batch, W_enc, b_enc, W_self0, W_nbr0, b0, W_self1, W_nbr1, b1, W_self2, W_nbr2, b2, W_dec, b_dec):
    raise NotImplementedError("write your pallas kernel here")


# speedup vs baseline: 6.9851x; 6.9851x over previous
import functools

import jax
import jax.numpy as jnp
from jax import lax
from jax.experimental import pallas as pl
from jax.experimental.pallas import tpu as pltpu
from jax.experimental.pallas import tpu_sc as plsc

N = 10000
E = 320000
H = 128
G = 64
OUT = 10
NC = 2
NS = 16
NW = NC * NS
EPT = E // NW
C = 80
NCHUNK = EPT // C
RPT = N // NS
RCH = 25
BN = 1000
NB = N // BN


def _agg_body(hf, src_hbm, dst_hbm, agg_hbm, src_v, dst_v, rows_v, zbuf,
              agg_sh, sem):
    c = lax.axis_index("c")
    s = lax.axis_index("s")
    w = c * NS + s

    def _zb(i, carry):
        for j in range(H // 16):
            zbuf[i, pl.ds(j * 16, 16)] = jnp.zeros((16,), jnp.float32)
        return carry
    lax.fori_loop(0, RCH, _zb, 0)
    for k in range(RPT // RCH):
        pltpu.sync_copy(zbuf, agg_sh.at[pl.ds(s * RPT + k * RCH, RCH)])
    plsc.subcore_barrier()

    pltpu.sync_copy(src_hbm.at[w], src_v)
    pltpu.sync_copy(dst_hbm.at[w], dst_v)

    def _edge(i, carry):
        pltpu.async_copy(hf.at[src_v.at[pl.ds(i * C, C)]], rows_v, sem).wait()
        pltpu.sync_copy(rows_v, agg_sh.at[dst_v.at[i]], add=True)
        return carry
    lax.fori_loop(0, NCHUNK, _edge, 0)
    plsc.subcore_barrier()

    pltpu.sync_copy(agg_sh.at[pl.ds(s * RPT, RPT)], agg_hbm.at[w])


_agg = pl.kernel(
    _agg_body,
    out_type=jax.ShapeDtypeStruct((NW, RPT, H), jnp.float32),
    mesh=plsc.VectorSubcoreMesh(core_axis_name="c", subcore_axis_name="s",
                                num_cores=NC, num_subcores=NS),
    scratch_types=[
        pltpu.VMEM((EPT,), jnp.int32),
        pltpu.VMEM((NCHUNK, C), jnp.int32),
        pltpu.VMEM((C, H), jnp.float32),
        pltpu.VMEM((RCH, H), jnp.float32),
        pltpu.VMEM_SHARED((N, H), jnp.float32),
        pltpu.SemaphoreType.DMA,
    ],
)


def _enc_body(x_ref, w_ref, b_ref, o_ref):
    o_ref[...] = jnp.dot(x_ref[...], w_ref[...],
                         preferred_element_type=jnp.float32) + b_ref[...]


_enc = pl.pallas_call(
    _enc_body,
    grid=(NB,),
    in_specs=[
        pl.BlockSpec((BN, H), lambda i: (i, 0)),
        pl.BlockSpec((H, H), lambda i: (0, 0)),
        pl.BlockSpec((1, H), lambda i: (0, 0)),
    ],
    out_specs=pl.BlockSpec((BN, H), lambda i: (i, 0)),
    out_shape=jax.ShapeDtypeStruct((N, H), jnp.float32),
)


def _layer_body(h_ref, a_ref, ws_ref, wn_ref, b_ref, o_ref, *, relu):
    agg = a_ref[0] + a_ref[1]
    out = (jnp.dot(h_ref[...], ws_ref[...], preferred_element_type=jnp.float32)
           + jnp.dot(agg, wn_ref[...], preferred_element_type=jnp.float32)
           + b_ref[...])
    if relu:
        out = jnp.maximum(out, 0.0)
    o_ref[...] = out


def _make_layer(relu):
    return pl.pallas_call(
        functools.partial(_layer_body, relu=relu),
        grid=(NB,),
        in_specs=[
            pl.BlockSpec((BN, H), lambda i: (i, 0)),
            pl.BlockSpec((NC, BN, H), lambda i: (0, i, 0)),
            pl.BlockSpec((H, H), lambda i: (0, 0)),
            pl.BlockSpec((H, H), lambda i: (0, 0)),
            pl.BlockSpec((1, H), lambda i: (0, 0)),
        ],
        out_specs=pl.BlockSpec((BN, H), lambda i: (i, 0)),
        out_shape=jax.ShapeDtypeStruct((N, H), jnp.float32),
    )


_layer_relu = _make_layer(True)
_layer_last = _make_layer(False)


def _pool_body(h_ref, batch_ref, wd_ref, bd_ref, o_ref):
    i = pl.program_id(0)
    bvec = batch_ref[0]
    rows = lax.broadcasted_iota(jnp.int32, (G, BN), 0)
    m = (rows == bvec).astype(jnp.float32)
    g = jnp.dot(m, h_ref[...], preferred_element_type=jnp.float32)
    contrib = jnp.dot(g, wd_ref[...], preferred_element_type=jnp.float32)

    @pl.when(i == 0)
    def _():
        o_ref[...] = contrib + bd_ref[...]

    @pl.when(i > 0)
    def _():
        o_ref[...] += contrib


_pool = pl.pallas_call(
    _pool_body,
    grid=(NB,),
    in_specs=[
        pl.BlockSpec((BN, H), lambda i: (i, 0)),
        pl.BlockSpec((1, 1, BN), lambda i: (i, 0, 0)),
        pl.BlockSpec((H, OUT), lambda i: (0, 0)),
        pl.BlockSpec((1, OUT), lambda i: (0, 0)),
    ],
    out_specs=pl.BlockSpec((G, OUT), lambda i: (0, 0)),
    out_shape=jax.ShapeDtypeStruct((G, OUT), jnp.float32),
)


def kernel(x, edge_attr, edge_index, batch, W_enc, b_enc, W_self0, W_nbr0, b0,
           W_self1, W_nbr1, b1, W_self2, W_nbr2, b2, W_dec, b_dec):
    src2 = edge_index[0].reshape(NW, EPT)
    dst3 = edge_index[1].reshape(NW, NCHUNK, C)
    batch3 = batch.reshape(NB, 1, BN)

    h = _enc(x, W_enc, b_enc.reshape(1, H))
    for Ws, Wn, b, lyr in ((W_self0, W_nbr0, b0, _layer_relu),
                           (W_self1, W_nbr1, b1, _layer_relu),
                           (W_self2, W_nbr2, b2, _layer_last)):
        aggf = _agg(h, src2, dst3)
        h = lyr(h, aggf.reshape(NC, N, H), Ws, Wn, b.reshape(1, H))
    return _pool(h, batch3, W_dec, b_dec.reshape(1, OUT))


# speedup vs baseline: 10.9840x; 1.5725x over previous
import functools

import jax
import jax.numpy as jnp
from jax import lax
from jax.experimental import pallas as pl
from jax.experimental.pallas import tpu as pltpu
from jax.experimental.pallas import tpu_sc as plsc

N = 10000
E = 320000
H = 128
G = 64
OUT = 10
NC = 2
NS = 16
NW = NC * NS
EPT = E // NW
C = 80
NCHUNK = EPT // C
RPT = N // NS
RCH = 25
BN = 1000
NB = N // BN


def _agg_body(hf, src_hbm, dst_hbm, agg_hbm, src_v, dst_v, rows_v, zbuf,
              agg_sh, sem0, sem1):
    c = lax.axis_index("c")
    s = lax.axis_index("s")
    w = c * NS + s

    def _zb(i, carry):
        for j in range(H // 16):
            zbuf[i, pl.ds(j * 16, 16)] = jnp.zeros((16,), jnp.float32)
        return carry
    lax.fori_loop(0, RCH, _zb, 0)
    for k in range(RPT // RCH):
        pltpu.sync_copy(zbuf, agg_sh.at[pl.ds(s * RPT + k * RCH, RCH)])
    plsc.subcore_barrier()

    pltpu.sync_copy(src_hbm.at[w], src_v)
    pltpu.sync_copy(dst_hbm.at[w], dst_v)

    sems = (sem0, sem1)

    def _gather(i, b):
        pltpu.async_copy(hf.at[src_v.at[pl.ds(i * C, C)]], rows_v.at[b],
                         sems[b])

    def _wait_scatter(i, b):
        pltpu.make_async_copy(hf.at[src_v.at[pl.ds(0, C)]], rows_v.at[b],
                              sems[b]).wait()
        pltpu.sync_copy(rows_v.at[b], agg_sh.at[dst_v.at[i]], add=True)

    _gather(0, 0)
    def _edge2(k, carry):
        i = 2 * k
        _gather(i + 1, 1)
        _wait_scatter(i, 0)
        _gather(i + 2, 0)
        _wait_scatter(i + 1, 1)
        return carry
    lax.fori_loop(0, (NCHUNK - 1) // 2, _edge2, 0)
    _wait_scatter(NCHUNK - 1, 0)
    plsc.subcore_barrier()

    pltpu.sync_copy(agg_sh.at[pl.ds(s * RPT, RPT)], agg_hbm.at[w])


_agg = pl.kernel(
    _agg_body,
    out_type=jax.ShapeDtypeStruct((NW, RPT, H), jnp.float32),
    mesh=plsc.VectorSubcoreMesh(core_axis_name="c", subcore_axis_name="s",
                                num_cores=NC, num_subcores=NS),
    scratch_types=[
        pltpu.VMEM((EPT,), jnp.int32),
        pltpu.VMEM((NCHUNK, C), jnp.int32),
        pltpu.VMEM((2, C, H), jnp.float32),
        pltpu.VMEM((RCH, H), jnp.float32),
        pltpu.VMEM_SHARED((N, H), jnp.float32),
        pltpu.SemaphoreType.DMA,
        pltpu.SemaphoreType.DMA,
    ],
)


def _enc_body(x_ref, w_ref, b_ref, o_ref):
    o_ref[...] = jnp.dot(x_ref[...], w_ref[...],
                         preferred_element_type=jnp.float32) + b_ref[...]


_enc = pl.pallas_call(
    _enc_body,
    grid=(NB,),
    in_specs=[
        pl.BlockSpec((BN, H), lambda i: (i, 0)),
        pl.BlockSpec((H, H), lambda i: (0, 0)),
        pl.BlockSpec((1, H), lambda i: (0, 0)),
    ],
    out_specs=pl.BlockSpec((BN, H), lambda i: (i, 0)),
    out_shape=jax.ShapeDtypeStruct((N, H), jnp.float32),
)


def _layer_body(h_ref, a_ref, ws_ref, wn_ref, b_ref, o_ref, *, relu):
    agg = a_ref[0] + a_ref[1]
    out = (jnp.dot(h_ref[...], ws_ref[...], preferred_element_type=jnp.float32)
           + jnp.dot(agg, wn_ref[...], preferred_element_type=jnp.float32)
           + b_ref[...])
    if relu:
        out = jnp.maximum(out, 0.0)
    o_ref[...] = out


def _make_layer(relu):
    return pl.pallas_call(
        functools.partial(_layer_body, relu=relu),
        grid=(NB,),
        in_specs=[
            pl.BlockSpec((BN, H), lambda i: (i, 0)),
            pl.BlockSpec((NC, BN, H), lambda i: (0, i, 0)),
            pl.BlockSpec((H, H), lambda i: (0, 0)),
            pl.BlockSpec((H, H), lambda i: (0, 0)),
            pl.BlockSpec((1, H), lambda i: (0, 0)),
        ],
        out_specs=pl.BlockSpec((BN, H), lambda i: (i, 0)),
        out_shape=jax.ShapeDtypeStruct((N, H), jnp.float32),
    )


_layer_relu = _make_layer(True)
_layer_last = _make_layer(False)


def _pool_body(h_ref, batch_ref, wd_ref, bd_ref, o_ref):
    i = pl.program_id(0)
    bvec = batch_ref[0]
    rows = lax.broadcasted_iota(jnp.int32, (G, BN), 0)
    m = (rows == bvec).astype(jnp.float32)
    g = jnp.dot(m, h_ref[...], preferred_element_type=jnp.float32)
    contrib = jnp.dot(g, wd_ref[...], preferred_element_type=jnp.float32)

    @pl.when(i == 0)
    def _():
        o_ref[...] = contrib + bd_ref[...]

    @pl.when(i > 0)
    def _():
        o_ref[...] += contrib


_pool = pl.pallas_call(
    _pool_body,
    grid=(NB,),
    in_specs=[
        pl.BlockSpec((BN, H), lambda i: (i, 0)),
        pl.BlockSpec((1, 1, BN), lambda i: (i, 0, 0)),
        pl.BlockSpec((H, OUT), lambda i: (0, 0)),
        pl.BlockSpec((1, OUT), lambda i: (0, 0)),
    ],
    out_specs=pl.BlockSpec((G, OUT), lambda i: (0, 0)),
    out_shape=jax.ShapeDtypeStruct((G, OUT), jnp.float32),
)


def kernel(x, edge_attr, edge_index, batch, W_enc, b_enc, W_self0, W_nbr0, b0,
           W_self1, W_nbr1, b1, W_self2, W_nbr2, b2, W_dec, b_dec):
    src2 = edge_index[0].reshape(NW, EPT)
    dst3 = edge_index[1].reshape(NW, NCHUNK, C)
    batch3 = batch.reshape(NB, 1, BN)

    h = _enc(x, W_enc, b_enc.reshape(1, H))
    for Ws, Wn, b, lyr in ((W_self0, W_nbr0, b0, _layer_relu),
                           (W_self1, W_nbr1, b1, _layer_relu),
                           (W_self2, W_nbr2, b2, _layer_last)):
        aggf = _agg(h, src2, dst3)
        h = lyr(h, aggf.reshape(NC, N, H), Ws, Wn, b.reshape(1, H))
    return _pool(h, batch3, W_dec, b_dec.reshape(1, OUT))


# speedup vs baseline: 11.7137x; 1.0664x over previous
import functools

import jax
import jax.numpy as jnp
from jax import lax
from jax.experimental import pallas as pl
from jax.experimental.pallas import tpu as pltpu
from jax.experimental.pallas import tpu_sc as plsc

N = 10000
E = 320000
H = 128
G = 64
OUT = 10
NC = 2
NS = 16
NW = NC * NS
EPT = E // NW
C = 100
NCHUNK = EPT // C
SUP = 5
NSUP = NCHUNK // SUP
NRING = 3
RPT = N // NS
BN = 1000
NB = N // BN


def _agg_body(hf, src_hbm, dst_hbm, agg_hbm, srcb, dstb, rows_v, agg_sh,
              gsem):
    c = lax.axis_index("c")
    s = lax.axis_index("s")
    w = c * NS + s

    def _zr(i, carry):
        for j in range(H // 16):
            rows_v[0, i, pl.ds(j * 16, 16)] = jnp.zeros((16,), jnp.float32)
        return carry
    lax.fori_loop(0, C, _zr, 0)
    for k in range(RPT // C):
        pltpu.sync_copy(rows_v.at[0], agg_sh.at[pl.ds(s * RPT + k * C, C)])
    _tail = RPT - (RPT // C) * C
    if _tail:
        pltpu.sync_copy(rows_v.at[0, pl.ds(0, _tail)],
                        agg_sh.at[pl.ds(s * RPT + (RPT // C) * C, _tail)])
    plsc.subcore_barrier()

    pltpu.sync_copy(src_hbm.at[w, 0], srcb.at[0])
    pltpu.sync_copy(dst_hbm.at[w, 0], dstb.at[0])
    pltpu.sync_copy(src_hbm.at[w, 1], srcb.at[1])
    pltpu.sync_copy(dst_hbm.at[w, 1], dstb.at[1])

    def _issue(i):
        m = lax.div(i, SUP)
        j = lax.rem(i, SUP)
        pltpu.async_copy(hf.at[srcb.at[lax.rem(m, 2), j]],
                         rows_v.at[lax.rem(i, NRING)], gsem)

    _issue(0)
    _issue(1)

    def _sup(m, carry):
        base = m * SUP
        for j in range(SUP):
            i = base + j
            nxt = i + 2

            @pl.when(nxt < NCHUNK)
            def _():
                _issue(nxt)

            slot = lax.rem(i, NRING)
            pltpu.make_async_copy(hf.at[srcb.at[0, 0]], rows_v.at[slot],
                                  gsem).wait()
            pltpu.sync_copy(rows_v.at[slot],
                            agg_sh.at[dstb.at[lax.rem(m, 2), j]], add=True)

        @pl.when(m + 2 < NSUP)
        def _():
            pltpu.sync_copy(src_hbm.at[w, m + 2], srcb.at[lax.rem(m, 2)])
            pltpu.sync_copy(dst_hbm.at[w, m + 2], dstb.at[lax.rem(m, 2)])
        return carry
    lax.fori_loop(0, NSUP, _sup, 0)
    plsc.subcore_barrier()

    pltpu.sync_copy(agg_sh.at[pl.ds(s * RPT, RPT)], agg_hbm.at[w])


_agg = pl.kernel(
    _agg_body,
    out_type=jax.ShapeDtypeStruct((NW, RPT, H), jnp.float32),
    mesh=plsc.VectorSubcoreMesh(core_axis_name="c", subcore_axis_name="s",
                                num_cores=NC, num_subcores=NS),
    scratch_types=[
        pltpu.VMEM((2, SUP, C), jnp.int32),
        pltpu.VMEM((2, SUP, C), jnp.int32),
        pltpu.VMEM((NRING, C, H), jnp.float32),
        pltpu.VMEM_SHARED((N, H), jnp.float32),
        pltpu.SemaphoreType.DMA,
    ],
)


def _enc_body(x_ref, w_ref, b_ref, o_ref):
    o_ref[...] = jnp.dot(x_ref[...], w_ref[...],
                         preferred_element_type=jnp.float32) + b_ref[...]


_enc = pl.pallas_call(
    _enc_body,
    grid=(NB,),
    in_specs=[
        pl.BlockSpec((BN, H), lambda i: (i, 0)),
        pl.BlockSpec((H, H), lambda i: (0, 0)),
        pl.BlockSpec((1, H), lambda i: (0, 0)),
    ],
    out_specs=pl.BlockSpec((BN, H), lambda i: (i, 0)),
    out_shape=jax.ShapeDtypeStruct((N, H), jnp.float32),
)


def _layer_body(h_ref, a_ref, ws_ref, wn_ref, b_ref, o_ref, *, relu):
    agg = a_ref[0] + a_ref[1]
    out = (jnp.dot(h_ref[...], ws_ref[...], preferred_element_type=jnp.float32)
           + jnp.dot(agg, wn_ref[...], preferred_element_type=jnp.float32)
           + b_ref[...])
    if relu:
        out = jnp.maximum(out, 0.0)
    o_ref[...] = out


def _make_layer(relu):
    return pl.pallas_call(
        functools.partial(_layer_body, relu=relu),
        grid=(NB,),
        in_specs=[
            pl.BlockSpec((BN, H), lambda i: (i, 0)),
            pl.BlockSpec((NC, BN, H), lambda i: (0, i, 0)),
            pl.BlockSpec((H, H), lambda i: (0, 0)),
            pl.BlockSpec((H, H), lambda i: (0, 0)),
            pl.BlockSpec((1, H), lambda i: (0, 0)),
        ],
        out_specs=pl.BlockSpec((BN, H), lambda i: (i, 0)),
        out_shape=jax.ShapeDtypeStruct((N, H), jnp.float32),
    )


_layer_relu = _make_layer(True)
_layer_last = _make_layer(False)


def _pool_body(h_ref, batch_ref, wd_ref, bd_ref, o_ref):
    i = pl.program_id(0)
    bvec = batch_ref[0]
    rows = lax.broadcasted_iota(jnp.int32, (G, BN), 0)
    m = (rows == bvec).astype(jnp.float32)
    g = jnp.dot(m, h_ref[...], preferred_element_type=jnp.float32)
    contrib = jnp.dot(g, wd_ref[...], preferred_element_type=jnp.float32)

    @pl.when(i == 0)
    def _():
        o_ref[...] = contrib + bd_ref[...]

    @pl.when(i > 0)
    def _():
        o_ref[...] += contrib


_pool = pl.pallas_call(
    _pool_body,
    grid=(NB,),
    in_specs=[
        pl.BlockSpec((BN, H), lambda i: (i, 0)),
        pl.BlockSpec((1, 1, BN), lambda i: (i, 0, 0)),
        pl.BlockSpec((H, OUT), lambda i: (0, 0)),
        pl.BlockSpec((1, OUT), lambda i: (0, 0)),
    ],
    out_specs=pl.BlockSpec((G, OUT), lambda i: (0, 0)),
    out_shape=jax.ShapeDtypeStruct((G, OUT), jnp.float32),
)


def kernel(x, edge_attr, edge_index, batch, W_enc, b_enc, W_self0, W_nbr0, b0,
           W_self1, W_nbr1, b1, W_self2, W_nbr2, b2, W_dec, b_dec):
    src4 = edge_index[0].reshape(NW, NSUP, SUP, C)
    dst4 = edge_index[1].reshape(NW, NSUP, SUP, C)
    batch3 = batch.reshape(NB, 1, BN)

    h = _enc(x, W_enc, b_enc.reshape(1, H))
    for Ws, Wn, b, lyr in ((W_self0, W_nbr0, b0, _layer_relu),
                           (W_self1, W_nbr1, b1, _layer_relu),
                           (W_self2, W_nbr2, b2, _layer_last)):
        aggf = _agg(h, src4, dst4)
        h = lyr(h, aggf.reshape(NC, N, H), Ws, Wn, b.reshape(1, H))
    return _pool(h, batch3, W_dec, b_dec.reshape(1, OUT))


# speedup vs baseline: 11.8058x; 1.0079x over previous
import functools

import jax
import jax.numpy as jnp
from jax import lax
from jax.experimental import pallas as pl
from jax.experimental.pallas import tpu as pltpu
from jax.experimental.pallas import tpu_sc as plsc

N = 10000
E = 320000
H = 128
G = 64
OUT = 10
NC = 2
NS = 16
NW = NC * NS
EPT = E // NW
C = 100
NCHUNK = EPT // C
SUP = 5
NSUP = NCHUNK // SUP
NRING = 3
RPT = N // NS
BN = 1000
NB = N // BN


def _agg_body(hf, src_hbm, dst_hbm, agg_hbm, srcb, dstb, rows_v, agg_sh,
              gsem):
    c = lax.axis_index("c")
    s = lax.axis_index("s")
    w = c * NS + s

    def _zr(i, carry):
        for j in range(H // 16):
            rows_v[0, i, pl.ds(j * 16, 16)] = jnp.zeros((16,), jnp.float32)
        return carry
    lax.fori_loop(0, C, _zr, 0)
    for k in range(RPT // C):
        pltpu.sync_copy(rows_v.at[0], agg_sh.at[pl.ds(s * RPT + k * C, C)])
    _tail = RPT - (RPT // C) * C
    if _tail:
        pltpu.sync_copy(rows_v.at[0, pl.ds(0, _tail)],
                        agg_sh.at[pl.ds(s * RPT + (RPT // C) * C, _tail)])
    plsc.subcore_barrier()

    pltpu.sync_copy(src_hbm.at[w, 0], srcb.at[0])
    pltpu.sync_copy(dst_hbm.at[w, 0], dstb.at[0])
    pltpu.sync_copy(src_hbm.at[w, 1], srcb.at[1])
    pltpu.sync_copy(dst_hbm.at[w, 1], dstb.at[1])

    def _issue(i):
        m = lax.div(i, SUP)
        j = lax.rem(i, SUP)
        pltpu.async_copy(hf.at[srcb.at[lax.rem(m, 2), j]],
                         rows_v.at[lax.rem(i, NRING)], gsem)

    _issue(0)
    _issue(1)

    def _sup(m, carry):
        base = m * SUP
        for j in range(SUP):
            i = base + j
            nxt = i + 2

            @pl.when(nxt < NCHUNK)
            def _():
                _issue(nxt)

            slot = lax.rem(i, NRING)
            pltpu.make_async_copy(hf.at[srcb.at[0, 0]], rows_v.at[slot],
                                  gsem).wait()
            pltpu.sync_copy(rows_v.at[slot],
                            agg_sh.at[dstb.at[lax.rem(m, 2), j]], add=True)

        @pl.when(m + 2 < NSUP)
        def _():
            pltpu.sync_copy(src_hbm.at[w, m + 2], srcb.at[lax.rem(m, 2)])
            pltpu.sync_copy(dst_hbm.at[w, m + 2], dstb.at[lax.rem(m, 2)])
        return carry
    lax.fori_loop(0, NSUP, _sup, 0)
    plsc.subcore_barrier()

    pltpu.sync_copy(agg_sh.at[pl.ds(s * RPT, RPT)], agg_hbm.at[w])


_agg = pl.kernel(
    _agg_body,
    out_type=jax.ShapeDtypeStruct((NW, RPT, H), jnp.float32),
    mesh=plsc.VectorSubcoreMesh(core_axis_name="c", subcore_axis_name="s",
                                num_cores=NC, num_subcores=NS),
    scratch_types=[
        pltpu.VMEM((2, SUP, C), jnp.int32),
        pltpu.VMEM((2, SUP, C), jnp.int32),
        pltpu.VMEM((NRING, C, H), jnp.float32),
        pltpu.VMEM_SHARED((N, H), jnp.float32),
        pltpu.SemaphoreType.DMA,
    ],
)


def _enc_body(x_ref, we_ref, be_ref, ws_ref, b0_ref, h_ref, s_ref):
    h0 = jnp.dot(x_ref[...], we_ref[...],
                 preferred_element_type=jnp.float32) + be_ref[...]
    h_ref[...] = h0
    s_ref[...] = jnp.dot(h0, ws_ref[...],
                         preferred_element_type=jnp.float32) + b0_ref[...]


_enc = pl.pallas_call(
    _enc_body,
    grid=(NB,),
    in_specs=[
        pl.BlockSpec((BN, H), lambda i: (i, 0)),
        pl.BlockSpec((H, H), lambda i: (0, 0)),
        pl.BlockSpec((1, H), lambda i: (0, 0)),
        pl.BlockSpec((H, H), lambda i: (0, 0)),
        pl.BlockSpec((1, H), lambda i: (0, 0)),
    ],
    out_specs=[
        pl.BlockSpec((BN, H), lambda i: (i, 0)),
        pl.BlockSpec((BN, H), lambda i: (i, 0)),
    ],
    out_shape=[
        jax.ShapeDtypeStruct((N, H), jnp.float32),
        jax.ShapeDtypeStruct((N, H), jnp.float32),
    ],
)


def _combine_body(self_ref, a_ref, wn_ref, ws_ref, b_ref, h_ref, s_ref):
    agg = a_ref[0] + a_ref[1]
    hn = jnp.maximum(
        self_ref[...] + jnp.dot(agg, wn_ref[...],
                                preferred_element_type=jnp.float32), 0.0)
    h_ref[...] = hn
    s_ref[...] = jnp.dot(hn, ws_ref[...],
                         preferred_element_type=jnp.float32) + b_ref[...]


_combine = pl.pallas_call(
    _combine_body,
    grid=(NB,),
    in_specs=[
        pl.BlockSpec((BN, H), lambda i: (i, 0)),
        pl.BlockSpec((NC, BN, H), lambda i: (0, i, 0)),
        pl.BlockSpec((H, H), lambda i: (0, 0)),
        pl.BlockSpec((H, H), lambda i: (0, 0)),
        pl.BlockSpec((1, H), lambda i: (0, 0)),
    ],
    out_specs=[
        pl.BlockSpec((BN, H), lambda i: (i, 0)),
        pl.BlockSpec((BN, H), lambda i: (i, 0)),
    ],
    out_shape=[
        jax.ShapeDtypeStruct((N, H), jnp.float32),
        jax.ShapeDtypeStruct((N, H), jnp.float32),
    ],
)


def _last_body(self_ref, a_ref, wn_ref, batch_ref, wd_ref, bd_ref, o_ref):
    i = pl.program_id(0)
    agg = a_ref[0] + a_ref[1]
    h3 = self_ref[...] + jnp.dot(agg, wn_ref[...],
                                 preferred_element_type=jnp.float32)
    bvec = batch_ref[0]
    rows = lax.broadcasted_iota(jnp.int32, (G, BN), 0)
    m = (rows == bvec).astype(jnp.float32)
    g = jnp.dot(m, h3, preferred_element_type=jnp.float32)
    contrib = jnp.dot(g, wd_ref[...], preferred_element_type=jnp.float32)

    @pl.when(i == 0)
    def _():
        o_ref[...] = contrib + bd_ref[...]

    @pl.when(i > 0)
    def _():
        o_ref[...] += contrib


_last = pl.pallas_call(
    _last_body,
    grid=(NB,),
    in_specs=[
        pl.BlockSpec((BN, H), lambda i: (i, 0)),
        pl.BlockSpec((NC, BN, H), lambda i: (0, i, 0)),
        pl.BlockSpec((H, H), lambda i: (0, 0)),
        pl.BlockSpec((1, 1, BN), lambda i: (i, 0, 0)),
        pl.BlockSpec((H, OUT), lambda i: (0, 0)),
        pl.BlockSpec((1, OUT), lambda i: (0, 0)),
    ],
    out_specs=pl.BlockSpec((G, OUT), lambda i: (0, 0)),
    out_shape=jax.ShapeDtypeStruct((G, OUT), jnp.float32),
)


def kernel(x, edge_attr, edge_index, batch, W_enc, b_enc, W_self0, W_nbr0, b0,
           W_self1, W_nbr1, b1, W_self2, W_nbr2, b2, W_dec, b_dec):
    src4 = edge_index[0].reshape(NW, NSUP, SUP, C)
    dst4 = edge_index[1].reshape(NW, NSUP, SUP, C)
    batch3 = batch.reshape(NB, 1, BN)

    h, self0 = _enc(x, W_enc, b_enc.reshape(1, H), W_self0, b0.reshape(1, H))
    a0 = _agg(h, src4, dst4).reshape(NC, N, H)
    h, self1 = _combine(self0, a0, W_nbr0, W_self1, b1.reshape(1, H))
    a1 = _agg(h, src4, dst4).reshape(NC, N, H)
    h, self2 = _combine(self1, a1, W_nbr1, W_self2, b2.reshape(1, H))
    a2 = _agg(h, src4, dst4).reshape(NC, N, H)
    return _last(self2, a2, W_nbr2, batch3, W_dec, b_dec.reshape(1, OUT))


# speedup vs baseline: 12.9016x; 1.0928x over previous
import functools

import jax
import jax.numpy as jnp
from jax import lax
from jax.experimental import pallas as pl
from jax.experimental.pallas import tpu as pltpu
from jax.experimental.pallas import tpu_sc as plsc

N = 10000
E = 320000
H = 128
G = 64
OUT = 10
NC = 2
NS = 16
NW = NC * NS
EPT = E // NW
C = 100
NCHUNK = EPT // C
SUP = 5
NSUP = NCHUNK // SUP
NRING = 3
RPT = N // NS
WRT = 624
BN = 1000
NB = N // BN


def _agg_body(hf, ei_hbm, agg_hbm, srcb, dstb, rows_v, agg_sh, gsem):
    c = lax.axis_index("c")
    s = lax.axis_index("s")
    w = c * NS + s

    def _zr(i, carry):
        for j in range(H // 16):
            rows_v[0, i, pl.ds(j * 16, 16)] = jnp.zeros((16,), jnp.float32)
        return carry
    lax.fori_loop(0, C, _zr, 0)
    for k in range(RPT // C):
        pltpu.sync_copy(rows_v.at[0], agg_sh.at[pl.ds(s * RPT + k * C, C)])
    _tail = RPT - (RPT // C) * C
    if _tail:
        pltpu.sync_copy(rows_v.at[0, pl.ds(0, _tail)],
                        agg_sh.at[pl.ds(s * RPT + (RPT // C) * C, _tail)])
    plsc.subcore_barrier()

    pltpu.sync_copy(ei_hbm.at[0, w, 0], srcb.at[0])
    pltpu.sync_copy(ei_hbm.at[1, w, 0], dstb.at[0])
    pltpu.sync_copy(ei_hbm.at[0, w, 1], srcb.at[1])
    pltpu.sync_copy(ei_hbm.at[1, w, 1], dstb.at[1])

    def _issue(i):
        m = lax.div(i, SUP)
        j = lax.rem(i, SUP)
        pltpu.async_copy(hf.at[srcb.at[lax.rem(m, 2), j]],
                         rows_v.at[lax.rem(i, NRING)], gsem)

    _issue(0)
    _issue(1)

    def _sup(m, carry):
        base = m * SUP
        for j in range(SUP):
            i = base + j
            nxt = i + 2

            @pl.when(nxt < NCHUNK)
            def _():
                _issue(nxt)

            slot = lax.rem(i, NRING)
            pltpu.make_async_copy(hf.at[srcb.at[0, 0]], rows_v.at[slot],
                                  gsem).wait()
            pltpu.sync_copy(rows_v.at[slot],
                            agg_sh.at[dstb.at[lax.rem(m, 2), j]], add=True)

        @pl.when(m + 2 < NSUP)
        def _():
            pltpu.sync_copy(ei_hbm.at[0, w, m + 2], srcb.at[lax.rem(m, 2)])
            pltpu.sync_copy(ei_hbm.at[1, w, m + 2], dstb.at[lax.rem(m, 2)])
        return carry
    lax.fori_loop(0, NSUP, _sup, 0)
    plsc.subcore_barrier()

    @pl.when(s < NS - 1)
    def _():
        pltpu.sync_copy(agg_sh.at[pl.ds(s * WRT, WRT)],
                        agg_hbm.at[c, pl.ds(s * WRT, WRT)])

    @pl.when(s == NS - 1)
    def _():
        pltpu.sync_copy(agg_sh.at[pl.ds((NS - 1) * WRT, N - (NS - 1) * WRT)],
                        agg_hbm.at[c, pl.ds((NS - 1) * WRT,
                                            N - (NS - 1) * WRT)])


_agg = pl.kernel(
    _agg_body,
    out_type=jax.ShapeDtypeStruct((NC, N, H), jnp.float32),
    mesh=plsc.VectorSubcoreMesh(core_axis_name="c", subcore_axis_name="s",
                                num_cores=NC, num_subcores=NS),
    scratch_types=[
        pltpu.VMEM((2, SUP, C), jnp.int32),
        pltpu.VMEM((2, SUP, C), jnp.int32),
        pltpu.VMEM((NRING, C, H), jnp.float32),
        pltpu.VMEM_SHARED((N, H), jnp.float32),
        pltpu.SemaphoreType.DMA,
    ],
)


def _enc_body(x_ref, we_ref, be_ref, ws_ref, b0_ref, h_ref, s_ref):
    h0 = jnp.dot(x_ref[...], we_ref[...],
                 preferred_element_type=jnp.float32) + be_ref[...]
    h_ref[...] = h0
    s_ref[...] = jnp.dot(h0, ws_ref[...],
                         preferred_element_type=jnp.float32) + b0_ref[...]


_enc = pl.pallas_call(
    _enc_body,
    grid=(NB,),
    in_specs=[
        pl.BlockSpec((BN, H), lambda i: (i, 0)),
        pl.BlockSpec((H, H), lambda i: (0, 0)),
        pl.BlockSpec((1, H), lambda i: (0, 0)),
        pl.BlockSpec((H, H), lambda i: (0, 0)),
        pl.BlockSpec((1, H), lambda i: (0, 0)),
    ],
    out_specs=[
        pl.BlockSpec((BN, H), lambda i: (i, 0)),
        pl.BlockSpec((BN, H), lambda i: (i, 0)),
    ],
    out_shape=[
        jax.ShapeDtypeStruct((N, H), jnp.float32),
        jax.ShapeDtypeStruct((N, H), jnp.float32),
    ],
)


def _combine_body(self_ref, a_ref, wn_ref, ws_ref, b_ref, h_ref, s_ref):
    agg = a_ref[0] + a_ref[1]
    hn = jnp.maximum(
        self_ref[...] + jnp.dot(agg, wn_ref[...],
                                preferred_element_type=jnp.float32), 0.0)
    h_ref[...] = hn
    s_ref[...] = jnp.dot(hn, ws_ref[...],
                         preferred_element_type=jnp.float32) + b_ref[...]


_combine = pl.pallas_call(
    _combine_body,
    grid=(NB,),
    in_specs=[
        pl.BlockSpec((BN, H), lambda i: (i, 0)),
        pl.BlockSpec((NC, BN, H), lambda i: (0, i, 0)),
        pl.BlockSpec((H, H), lambda i: (0, 0)),
        pl.BlockSpec((H, H), lambda i: (0, 0)),
        pl.BlockSpec((1, H), lambda i: (0, 0)),
    ],
    out_specs=[
        pl.BlockSpec((BN, H), lambda i: (i, 0)),
        pl.BlockSpec((BN, H), lambda i: (i, 0)),
    ],
    out_shape=[
        jax.ShapeDtypeStruct((N, H), jnp.float32),
        jax.ShapeDtypeStruct((N, H), jnp.float32),
    ],
)


def _last_body(self_ref, a_ref, wn_ref, batch_ref, wd_ref, bd_ref, o_ref):
    i = pl.program_id(0)
    agg = a_ref[0] + a_ref[1]
    h3 = self_ref[...] + jnp.dot(agg, wn_ref[...],
                                 preferred_element_type=jnp.float32)
    bvec = batch_ref[0]
    rows = lax.broadcasted_iota(jnp.int32, (G, BN), 0)
    m = (rows == bvec).astype(jnp.float32)
    g = jnp.dot(m, h3, preferred_element_type=jnp.float32)
    contrib = jnp.dot(g, wd_ref[...], preferred_element_type=jnp.float32)

    @pl.when(i == 0)
    def _():
        o_ref[...] = contrib + bd_ref[...]

    @pl.when(i > 0)
    def _():
        o_ref[...] += contrib


_last = pl.pallas_call(
    _last_body,
    grid=(NB,),
    in_specs=[
        pl.BlockSpec((BN, H), lambda i: (i, 0)),
        pl.BlockSpec((NC, BN, H), lambda i: (0, i, 0)),
        pl.BlockSpec((H, H), lambda i: (0, 0)),
        pl.BlockSpec((1, 1, BN), lambda i: (i, 0, 0)),
        pl.BlockSpec((H, OUT), lambda i: (0, 0)),
        pl.BlockSpec((1, OUT), lambda i: (0, 0)),
    ],
    out_specs=pl.BlockSpec((G, OUT), lambda i: (0, 0)),
    out_shape=jax.ShapeDtypeStruct((G, OUT), jnp.float32),
)


def kernel(x, edge_attr, edge_index, batch, W_enc, b_enc, W_self0, W_nbr0, b0,
           W_self1, W_nbr1, b1, W_self2, W_nbr2, b2, W_dec, b_dec):
    ei5 = edge_index.reshape(2, NW, NSUP, SUP, C)
    batch3 = batch.reshape(NB, 1, BN)

    h, self0 = _enc(x, W_enc, b_enc.reshape(1, H), W_self0, b0.reshape(1, H))
    a0 = _agg(h, ei5)
    h, self1 = _combine(self0, a0, W_nbr0, W_self1, b1.reshape(1, H))
    a1 = _agg(h, ei5)
    h, self2 = _combine(self1, a1, W_nbr1, W_self2, b2.reshape(1, H))
    a2 = _agg(h, ei5)
    return _last(self2, a2, W_nbr2, batch3, W_dec, b_dec.reshape(1, OUT))


# speedup vs baseline: 13.0104x; 1.0084x over previous
import functools

import jax
import jax.numpy as jnp
from jax import lax
from jax.experimental import pallas as pl
from jax.experimental.pallas import tpu as pltpu
from jax.experimental.pallas import tpu_sc as plsc

N = 10000
E = 320000
H = 128
G = 64
OUT = 10
NC = 2
NS = 16
NW = NC * NS
EPT = E // NW
C = 100
NCHUNK = EPT // C
SUP = 5
NSUP = NCHUNK // SUP
NRING = 3
RPT = N // NS
WRT = 624
BN = 1000
NB = N // BN


def _agg_body(hf, ei_hbm, agg_hbm, srcb, dstb, rows_v, agg_sh, gsem):
    c = lax.axis_index("c")
    s = lax.axis_index("s")
    w = c * NS + s

    def _zr(i, carry):
        for j in range(H // 16):
            rows_v[0, i, pl.ds(j * 16, 16)] = jnp.zeros((16,), jnp.float32)
        return carry
    lax.fori_loop(0, C, _zr, 0)
    for k in range(RPT // C):
        pltpu.sync_copy(rows_v.at[0], agg_sh.at[pl.ds(s * RPT + k * C, C)])
    _tail = RPT - (RPT // C) * C
    if _tail:
        pltpu.sync_copy(rows_v.at[0, pl.ds(0, _tail)],
                        agg_sh.at[pl.ds(s * RPT + (RPT // C) * C, _tail)])
    plsc.subcore_barrier()

    pltpu.sync_copy(ei_hbm.at[0, w, 0], srcb.at[0])
    pltpu.sync_copy(ei_hbm.at[1, w, 0], dstb.at[0])
    pltpu.sync_copy(ei_hbm.at[0, w, 1], srcb.at[1])
    pltpu.sync_copy(ei_hbm.at[1, w, 1], dstb.at[1])

    def _issue(i):
        m = lax.div(i, SUP)
        j = lax.rem(i, SUP)
        pltpu.async_copy(hf.at[srcb.at[lax.rem(m, 2), j]],
                         rows_v.at[lax.rem(i, NRING)], gsem)

    _issue(0)
    _issue(1)

    def _sup(m, carry):
        base = m * SUP
        for j in range(SUP):
            i = base + j
            nxt = i + 2

            @pl.when(nxt < NCHUNK)
            def _():
                _issue(nxt)

            slot = lax.rem(i, NRING)
            pltpu.make_async_copy(hf.at[srcb.at[0, 0]], rows_v.at[slot],
                                  gsem).wait()
            pltpu.sync_copy(rows_v.at[slot],
                            agg_sh.at[dstb.at[lax.rem(m, 2), j]], add=True)

        @pl.when(m + 2 < NSUP)
        def _():
            pltpu.sync_copy(ei_hbm.at[0, w, m + 2], srcb.at[lax.rem(m, 2)])
            pltpu.sync_copy(ei_hbm.at[1, w, m + 2], dstb.at[lax.rem(m, 2)])
        return carry
    lax.fori_loop(0, NSUP, _sup, 0)
    plsc.subcore_barrier()

    @pl.when(s < NS - 1)
    def _():
        pltpu.sync_copy(agg_sh.at[pl.ds(s * WRT, WRT)],
                        agg_hbm.at[c, pl.ds(s * WRT, WRT)])

    @pl.when(s == NS - 1)
    def _():
        pltpu.sync_copy(agg_sh.at[pl.ds((NS - 1) * WRT, N - (NS - 1) * WRT)],
                        agg_hbm.at[c, pl.ds((NS - 1) * WRT,
                                            N - (NS - 1) * WRT)])


_agg = pl.kernel(
    _agg_body,
    out_type=jax.ShapeDtypeStruct((NC, N, H), jnp.float32),
    mesh=plsc.VectorSubcoreMesh(core_axis_name="c", subcore_axis_name="s",
                                num_cores=NC, num_subcores=NS),
    scratch_types=[
        pltpu.VMEM((2, SUP, C), jnp.int32),
        pltpu.VMEM((2, SUP, C), jnp.int32),
        pltpu.VMEM((NRING, C, H), jnp.float32),
        pltpu.VMEM_SHARED((N, H), jnp.float32),
        pltpu.SemaphoreType.DMA,
    ],
)


def _enc_body(x_ref, we_ref, be_ref, h_ref):
    h_ref[...] = jnp.dot(x_ref[...], we_ref[...],
                         preferred_element_type=jnp.float32) + be_ref[...]


_enc = pl.pallas_call(
    _enc_body,
    grid=(NB,),
    in_specs=[
        pl.BlockSpec((BN, H), lambda i: (i, 0)),
        pl.BlockSpec((H, H), lambda i: (0, 0)),
        pl.BlockSpec((1, H), lambda i: (0, 0)),
    ],
    out_specs=pl.BlockSpec((BN, H), lambda i: (i, 0)),
    out_shape=jax.ShapeDtypeStruct((N, H), jnp.float32),
)


def _self_body(h_ref, ws_ref, b_ref, s_ref):
    s_ref[...] = jnp.dot(h_ref[...], ws_ref[...],
                         preferred_element_type=jnp.float32) + b_ref[...]


_self = pl.pallas_call(
    _self_body,
    grid=(NB,),
    in_specs=[
        pl.BlockSpec((BN, H), lambda i: (i, 0)),
        pl.BlockSpec((H, H), lambda i: (0, 0)),
        pl.BlockSpec((1, H), lambda i: (0, 0)),
    ],
    out_specs=pl.BlockSpec((BN, H), lambda i: (i, 0)),
    out_shape=jax.ShapeDtypeStruct((N, H), jnp.float32),
)


def _combine_body(self_ref, a_ref, wn_ref, h_ref):
    agg = a_ref[0] + a_ref[1]
    h_ref[...] = jnp.maximum(
        self_ref[...] + jnp.dot(agg, wn_ref[...],
                                preferred_element_type=jnp.float32), 0.0)


_combine = pl.pallas_call(
    _combine_body,
    grid=(NB,),
    in_specs=[
        pl.BlockSpec((BN, H), lambda i: (i, 0)),
        pl.BlockSpec((NC, BN, H), lambda i: (0, i, 0)),
        pl.BlockSpec((H, H), lambda i: (0, 0)),
    ],
    out_specs=pl.BlockSpec((BN, H), lambda i: (i, 0)),
    out_shape=jax.ShapeDtypeStruct((N, H), jnp.float32),
)


def _last_body(self_ref, a_ref, wn_ref, batch_ref, wd_ref, bd_ref, o_ref):
    i = pl.program_id(0)
    agg = a_ref[0] + a_ref[1]
    h3 = self_ref[...] + jnp.dot(agg, wn_ref[...],
                                 preferred_element_type=jnp.float32)
    bvec = batch_ref[0]
    rows = lax.broadcasted_iota(jnp.int32, (G, BN), 0)
    m = (rows == bvec).astype(jnp.float32)
    g = jnp.dot(m, h3, preferred_element_type=jnp.float32)
    contrib = jnp.dot(g, wd_ref[...], preferred_element_type=jnp.float32)

    @pl.when(i == 0)
    def _():
        o_ref[...] = contrib + bd_ref[...]

    @pl.when(i > 0)
    def _():
        o_ref[...] += contrib


_last = pl.pallas_call(
    _last_body,
    grid=(NB,),
    in_specs=[
        pl.BlockSpec((BN, H), lambda i: (i, 0)),
        pl.BlockSpec((NC, BN, H), lambda i: (0, i, 0)),
        pl.BlockSpec((H, H), lambda i: (0, 0)),
        pl.BlockSpec((1, 1, BN), lambda i: (i, 0, 0)),
        pl.BlockSpec((H, OUT), lambda i: (0, 0)),
        pl.BlockSpec((1, OUT), lambda i: (0, 0)),
    ],
    out_specs=pl.BlockSpec((G, OUT), lambda i: (0, 0)),
    out_shape=jax.ShapeDtypeStruct((G, OUT), jnp.float32),
)


def kernel(x, edge_attr, edge_index, batch, W_enc, b_enc, W_self0, W_nbr0, b0,
           W_self1, W_nbr1, b1, W_self2, W_nbr2, b2, W_dec, b_dec):
    ei5 = edge_index.reshape(2, NW, NSUP, SUP, C)
    batch3 = batch.reshape(NB, 1, BN)

    h = _enc(x, W_enc, b_enc.reshape(1, H))
    a0 = _agg(h, ei5)
    self0 = _self(h, W_self0, b0.reshape(1, H))
    h = _combine(self0, a0, W_nbr0)
    a1 = _agg(h, ei5)
    self1 = _self(h, W_self1, b1.reshape(1, H))
    h = _combine(self1, a1, W_nbr1)
    a2 = _agg(h, ei5)
    self2 = _self(h, W_self2, b2.reshape(1, H))
    return _last(self2, a2, W_nbr2, batch3, W_dec, b_dec.reshape(1, OUT))


# speedup vs baseline: 13.8246x; 1.0626x over previous
import functools

import jax
import jax.numpy as jnp
from jax import lax
from jax.experimental import pallas as pl
from jax.experimental.pallas import tpu as pltpu
from jax.experimental.pallas import tpu_sc as plsc

N = 10000
E = 320000
H = 128
G = 64
OUT = 10
NC = 2
NS = 16
NW = NC * NS
EPT = E // NW
C = 100
NCHUNK = EPT // C
SUP = 5
NSUP = NCHUNK // SUP
NRING = 3
RPT = N // NS
WRT = 624
BN = 1000
NB = N // BN


def _agg_body(hf, ei_hbm, agg_hbm, srcb, dstb, rows_v, agg_sh, gsem, isem):
    c = lax.axis_index("c")
    s = lax.axis_index("s")
    w = c * NS + s

    def _zr(i, carry):
        for j in range(H // 16):
            rows_v[0, i, pl.ds(j * 16, 16)] = jnp.zeros((16,), jnp.float32)
        return carry
    lax.fori_loop(0, C, _zr, 0)
    for k in range(RPT // C):
        pltpu.sync_copy(rows_v.at[0], agg_sh.at[pl.ds(s * RPT + k * C, C)])
    _tail = RPT - (RPT // C) * C
    if _tail:
        pltpu.sync_copy(rows_v.at[0, pl.ds(0, _tail)],
                        agg_sh.at[pl.ds(s * RPT + (RPT // C) * C, _tail)])
    plsc.subcore_barrier()

    pltpu.sync_copy(ei_hbm.at[0, w, 0], srcb.at[0])
    pltpu.sync_copy(ei_hbm.at[1, w, 0], dstb.at[0])
    pltpu.sync_copy(ei_hbm.at[0, w, 1], srcb.at[1])
    pltpu.sync_copy(ei_hbm.at[1, w, 1], dstb.at[1])

    def _issue(i):
        m = lax.div(i, SUP)
        j = lax.rem(i, SUP)
        pltpu.async_copy(hf.at[srcb.at[lax.rem(m, 2), j]],
                         rows_v.at[lax.rem(i, NRING)], gsem)

    _issue(0)
    _issue(1)

    def _sup(m, carry):
        @pl.when(jnp.logical_and(m >= 1, m < NSUP - 1))
        def _():
            pltpu.make_async_copy(ei_hbm.at[0, w, 0], srcb.at[0], isem).wait()
            pltpu.make_async_copy(ei_hbm.at[1, w, 0], dstb.at[0], isem).wait()

        base = m * SUP
        for j in range(SUP):
            i = base + j
            nxt = i + 2

            @pl.when(nxt < NCHUNK)
            def _():
                _issue(nxt)

            slot = lax.rem(i, NRING)
            pltpu.make_async_copy(hf.at[srcb.at[0, 0]], rows_v.at[slot],
                                  gsem).wait()
            pltpu.sync_copy(rows_v.at[slot],
                            agg_sh.at[dstb.at[lax.rem(m, 2), j]], add=True)

        @pl.when(m + 2 < NSUP)
        def _():
            pltpu.async_copy(ei_hbm.at[0, w, m + 2], srcb.at[lax.rem(m, 2)],
                             isem)
            pltpu.async_copy(ei_hbm.at[1, w, m + 2], dstb.at[lax.rem(m, 2)],
                             isem)
        return carry
    lax.fori_loop(0, NSUP, _sup, 0)
    plsc.subcore_barrier()

    @pl.when(s < NS - 1)
    def _():
        pltpu.sync_copy(agg_sh.at[pl.ds(s * WRT, WRT)],
                        agg_hbm.at[c, pl.ds(s * WRT, WRT)])

    @pl.when(s == NS - 1)
    def _():
        pltpu.sync_copy(agg_sh.at[pl.ds((NS - 1) * WRT, N - (NS - 1) * WRT)],
                        agg_hbm.at[c, pl.ds((NS - 1) * WRT,
                                            N - (NS - 1) * WRT)])


_agg = pl.kernel(
    _agg_body,
    out_type=jax.ShapeDtypeStruct((NC, N, H), jnp.float32),
    mesh=plsc.VectorSubcoreMesh(core_axis_name="c", subcore_axis_name="s",
                                num_cores=NC, num_subcores=NS),
    scratch_types=[
        pltpu.VMEM((2, SUP, C), jnp.int32),
        pltpu.VMEM((2, SUP, C), jnp.int32),
        pltpu.VMEM((NRING, C, H), jnp.float32),
        pltpu.VMEM_SHARED((N, H), jnp.float32),
        pltpu.SemaphoreType.DMA,
        pltpu.SemaphoreType.DMA,
    ],
)


def _enc_body(x_ref, we_ref, be_ref, h_ref):
    h_ref[...] = jnp.dot(x_ref[...], we_ref[...],
                         preferred_element_type=jnp.float32) + be_ref[...]


_enc = pl.pallas_call(
    _enc_body,
    grid=(NB,),
    in_specs=[
        pl.BlockSpec((BN, H), lambda i: (i, 0)),
        pl.BlockSpec((H, H), lambda i: (0, 0)),
        pl.BlockSpec((1, H), lambda i: (0, 0)),
    ],
    out_specs=pl.BlockSpec((BN, H), lambda i: (i, 0)),
    out_shape=jax.ShapeDtypeStruct((N, H), jnp.float32),
)


def _self_body(h_ref, ws_ref, b_ref, s_ref):
    s_ref[...] = jnp.dot(h_ref[...], ws_ref[...],
                         preferred_element_type=jnp.float32) + b_ref[...]


_self = pl.pallas_call(
    _self_body,
    grid=(NB,),
    in_specs=[
        pl.BlockSpec((BN, H), lambda i: (i, 0)),
        pl.BlockSpec((H, H), lambda i: (0, 0)),
        pl.BlockSpec((1, H), lambda i: (0, 0)),
    ],
    out_specs=pl.BlockSpec((BN, H), lambda i: (i, 0)),
    out_shape=jax.ShapeDtypeStruct((N, H), jnp.float32),
)


def _combine_body(self_ref, a_ref, wn_ref, h_ref):
    agg = a_ref[0] + a_ref[1]
    h_ref[...] = jnp.maximum(
        self_ref[...] + jnp.dot(agg, wn_ref[...],
                                preferred_element_type=jnp.float32), 0.0)


_combine = pl.pallas_call(
    _combine_body,
    grid=(NB,),
    in_specs=[
        pl.BlockSpec((BN, H), lambda i: (i, 0)),
        pl.BlockSpec((NC, BN, H), lambda i: (0, i, 0)),
        pl.BlockSpec((H, H), lambda i: (0, 0)),
    ],
    out_specs=pl.BlockSpec((BN, H), lambda i: (i, 0)),
    out_shape=jax.ShapeDtypeStruct((N, H), jnp.float32),
)


def _last_body(self_ref, a_ref, wn_ref, batch_ref, wd_ref, bd_ref, o_ref):
    i = pl.program_id(0)
    agg = a_ref[0] + a_ref[1]
    h3 = self_ref[...] + jnp.dot(agg, wn_ref[...],
                                 preferred_element_type=jnp.float32)
    bvec = batch_ref[0]
    rows = lax.broadcasted_iota(jnp.int32, (G, BN), 0)
    m = (rows == bvec).astype(jnp.float32)
    g = jnp.dot(m, h3, preferred_element_type=jnp.float32)
    contrib = jnp.dot(g, wd_ref[...], preferred_element_type=jnp.float32)

    @pl.when(i == 0)
    def _():
        o_ref[...] = contrib + bd_ref[...]

    @pl.when(i > 0)
    def _():
        o_ref[...] += contrib


_last = pl.pallas_call(
    _last_body,
    grid=(NB,),
    in_specs=[
        pl.BlockSpec((BN, H), lambda i: (i, 0)),
        pl.BlockSpec((NC, BN, H), lambda i: (0, i, 0)),
        pl.BlockSpec((H, H), lambda i: (0, 0)),
        pl.BlockSpec((1, 1, BN), lambda i: (i, 0, 0)),
        pl.BlockSpec((H, OUT), lambda i: (0, 0)),
        pl.BlockSpec((1, OUT), lambda i: (0, 0)),
    ],
    out_specs=pl.BlockSpec((G, OUT), lambda i: (0, 0)),
    out_shape=jax.ShapeDtypeStruct((G, OUT), jnp.float32),
)


def kernel(x, edge_attr, edge_index, batch, W_enc, b_enc, W_self0, W_nbr0, b0,
           W_self1, W_nbr1, b1, W_self2, W_nbr2, b2, W_dec, b_dec):
    ei5 = edge_index.reshape(2, NW, NSUP, SUP, C)
    batch3 = batch.reshape(NB, 1, BN)

    h = _enc(x, W_enc, b_enc.reshape(1, H))
    a0 = _agg(h, ei5)
    self0 = _self(h, W_self0, b0.reshape(1, H))
    h = _combine(self0, a0, W_nbr0)
    a1 = _agg(h, ei5)
    self1 = _self(h, W_self1, b1.reshape(1, H))
    h = _combine(self1, a1, W_nbr1)
    a2 = _agg(h, ei5)
    self2 = _self(h, W_self2, b2.reshape(1, H))
    return _last(self2, a2, W_nbr2, batch3, W_dec, b_dec.reshape(1, OUT))


# speedup vs baseline: 14.2419x; 1.0302x over previous
import functools

import jax
import jax.numpy as jnp
from jax import lax
from jax.experimental import pallas as pl
from jax.experimental.pallas import tpu as pltpu
from jax.experimental.pallas import tpu_sc as plsc

N = 10000
E = 320000
H = 128
G = 64
OUT = 10
NC = 2
NS = 16
NW = NC * NS
EPT = E // NW
C = 100
NCHUNK = EPT // C
SUP = 5
NSUP = NCHUNK // SUP
NRING = 3
RPT = N // NS
WRT = 624
BN = 2000
NB = N // BN


def _agg_body(hf, ei_hbm, agg_hbm, srcb, dstb, rows_v, agg_sh, gsem, isem):
    c = lax.axis_index("c")
    s = lax.axis_index("s")
    w = c * NS + s

    def _zr(i, carry):
        for j in range(H // 16):
            rows_v[0, i, pl.ds(j * 16, 16)] = jnp.zeros((16,), jnp.float32)
        return carry
    lax.fori_loop(0, C, _zr, 0)
    for k in range(RPT // C):
        pltpu.sync_copy(rows_v.at[0], agg_sh.at[pl.ds(s * RPT + k * C, C)])
    _tail = RPT - (RPT // C) * C
    if _tail:
        pltpu.sync_copy(rows_v.at[0, pl.ds(0, _tail)],
                        agg_sh.at[pl.ds(s * RPT + (RPT // C) * C, _tail)])
    plsc.subcore_barrier()

    pltpu.sync_copy(ei_hbm.at[0, w, 0], srcb.at[0])
    pltpu.sync_copy(ei_hbm.at[1, w, 0], dstb.at[0])
    pltpu.sync_copy(ei_hbm.at[0, w, 1], srcb.at[1])
    pltpu.sync_copy(ei_hbm.at[1, w, 1], dstb.at[1])

    def _issue(i):
        m = lax.div(i, SUP)
        j = lax.rem(i, SUP)
        pltpu.async_copy(hf.at[srcb.at[lax.rem(m, 2), j]],
                         rows_v.at[lax.rem(i, NRING)], gsem)

    _issue(0)
    _issue(1)

    def _sup(m, carry):
        @pl.when(jnp.logical_and(m >= 1, m < NSUP - 1))
        def _():
            pltpu.make_async_copy(ei_hbm.at[0, w, 0], srcb.at[0], isem).wait()
            pltpu.make_async_copy(ei_hbm.at[1, w, 0], dstb.at[0], isem).wait()

        base = m * SUP
        for j in range(SUP):
            i = base + j
            nxt = i + 2

            @pl.when(nxt < NCHUNK)
            def _():
                _issue(nxt)

            slot = lax.rem(i, NRING)
            pltpu.make_async_copy(hf.at[srcb.at[0, 0]], rows_v.at[slot],
                                  gsem).wait()
            pltpu.sync_copy(rows_v.at[slot],
                            agg_sh.at[dstb.at[lax.rem(m, 2), j]], add=True)

        @pl.when(m + 2 < NSUP)
        def _():
            pltpu.async_copy(ei_hbm.at[0, w, m + 2], srcb.at[lax.rem(m, 2)],
                             isem)
            pltpu.async_copy(ei_hbm.at[1, w, m + 2], dstb.at[lax.rem(m, 2)],
                             isem)
        return carry
    lax.fori_loop(0, NSUP, _sup, 0)
    plsc.subcore_barrier()

    @pl.when(s < NS - 1)
    def _():
        pltpu.sync_copy(agg_sh.at[pl.ds(s * WRT, WRT)],
                        agg_hbm.at[c, pl.ds(s * WRT, WRT)])

    @pl.when(s == NS - 1)
    def _():
        pltpu.sync_copy(agg_sh.at[pl.ds((NS - 1) * WRT, N - (NS - 1) * WRT)],
                        agg_hbm.at[c, pl.ds((NS - 1) * WRT,
                                            N - (NS - 1) * WRT)])


_agg = pl.kernel(
    _agg_body,
    out_type=jax.ShapeDtypeStruct((NC, N, H), jnp.float32),
    mesh=plsc.VectorSubcoreMesh(core_axis_name="c", subcore_axis_name="s",
                                num_cores=NC, num_subcores=NS),
    scratch_types=[
        pltpu.VMEM((2, SUP, C), jnp.int32),
        pltpu.VMEM((2, SUP, C), jnp.int32),
        pltpu.VMEM((NRING, C, H), jnp.float32),
        pltpu.VMEM_SHARED((N, H), jnp.float32),
        pltpu.SemaphoreType.DMA,
        pltpu.SemaphoreType.DMA,
    ],
)


def _enc_body(x_ref, we_ref, be_ref, h_ref):
    h_ref[...] = jnp.dot(x_ref[...], we_ref[...],
                         preferred_element_type=jnp.float32) + be_ref[...]


_enc = pl.pallas_call(
    _enc_body,
    grid=(NB,),
    in_specs=[
        pl.BlockSpec((BN, H), lambda i: (i, 0)),
        pl.BlockSpec((H, H), lambda i: (0, 0)),
        pl.BlockSpec((1, H), lambda i: (0, 0)),
    ],
    out_specs=pl.BlockSpec((BN, H), lambda i: (i, 0)),
    out_shape=jax.ShapeDtypeStruct((N, H), jnp.float32),
)


def _self_body(h_ref, ws_ref, b_ref, s_ref):
    s_ref[...] = jnp.dot(h_ref[...], ws_ref[...],
                         preferred_element_type=jnp.float32) + b_ref[...]


_self = pl.pallas_call(
    _self_body,
    grid=(NB,),
    in_specs=[
        pl.BlockSpec((BN, H), lambda i: (i, 0)),
        pl.BlockSpec((H, H), lambda i: (0, 0)),
        pl.BlockSpec((1, H), lambda i: (0, 0)),
    ],
    out_specs=pl.BlockSpec((BN, H), lambda i: (i, 0)),
    out_shape=jax.ShapeDtypeStruct((N, H), jnp.float32),
)


def _combine_body(self_ref, a_ref, wn_ref, h_ref):
    agg = a_ref[0] + a_ref[1]
    h_ref[...] = jnp.maximum(
        self_ref[...] + jnp.dot(agg, wn_ref[...],
                                preferred_element_type=jnp.float32), 0.0)


_combine = pl.pallas_call(
    _combine_body,
    grid=(NB,),
    in_specs=[
        pl.BlockSpec((BN, H), lambda i: (i, 0)),
        pl.BlockSpec((NC, BN, H), lambda i: (0, i, 0)),
        pl.BlockSpec((H, H), lambda i: (0, 0)),
    ],
    out_specs=pl.BlockSpec((BN, H), lambda i: (i, 0)),
    out_shape=jax.ShapeDtypeStruct((N, H), jnp.float32),
)


def _last_body(self_ref, a_ref, wn_ref, batch_ref, wd_ref, bd_ref, o_ref):
    i = pl.program_id(0)
    agg = a_ref[0] + a_ref[1]
    h3 = self_ref[...] + jnp.dot(agg, wn_ref[...],
                                 preferred_element_type=jnp.float32)
    bvec = batch_ref[0]
    rows = lax.broadcasted_iota(jnp.int32, (G, BN), 0)
    m = (rows == bvec).astype(jnp.float32)
    g = jnp.dot(m, h3, preferred_element_type=jnp.float32)
    contrib = jnp.dot(g, wd_ref[...], preferred_element_type=jnp.float32)

    @pl.when(i == 0)
    def _():
        o_ref[...] = contrib + bd_ref[...]

    @pl.when(i > 0)
    def _():
        o_ref[...] += contrib


_last = pl.pallas_call(
    _last_body,
    grid=(NB,),
    in_specs=[
        pl.BlockSpec((BN, H), lambda i: (i, 0)),
        pl.BlockSpec((NC, BN, H), lambda i: (0, i, 0)),
        pl.BlockSpec((H, H), lambda i: (0, 0)),
        pl.BlockSpec((1, 1, BN), lambda i: (i, 0, 0)),
        pl.BlockSpec((H, OUT), lambda i: (0, 0)),
        pl.BlockSpec((1, OUT), lambda i: (0, 0)),
    ],
    out_specs=pl.BlockSpec((G, OUT), lambda i: (0, 0)),
    out_shape=jax.ShapeDtypeStruct((G, OUT), jnp.float32),
)


def kernel(x, edge_attr, edge_index, batch, W_enc, b_enc, W_self0, W_nbr0, b0,
           W_self1, W_nbr1, b1, W_self2, W_nbr2, b2, W_dec, b_dec):
    ei5 = edge_index.reshape(2, NW, NSUP, SUP, C)
    batch3 = batch.reshape(NB, 1, BN)

    h = _enc(x, W_enc, b_enc.reshape(1, H))
    a0 = _agg(h, ei5)
    self0 = _self(h, W_self0, b0.reshape(1, H))
    h = _combine(self0, a0, W_nbr0)
    a1 = _agg(h, ei5)
    self1 = _self(h, W_self1, b1.reshape(1, H))
    h = _combine(self1, a1, W_nbr1)
    a2 = _agg(h, ei5)
    self2 = _self(h, W_self2, b2.reshape(1, H))
    return _last(self2, a2, W_nbr2, batch3, W_dec, b_dec.reshape(1, OUT))


# speedup vs baseline: 14.2503x; 1.0006x over previous
import jax
import jax.numpy as jnp
from jax import lax
from jax.experimental import pallas as pl
from jax.experimental.pallas import tpu as pltpu
from jax.experimental.pallas import tpu_sc as plsc

N = 10000
E = 320000
H = 128
G = 64
OUT = 10
NC = 2
NS = 16
NW = NC * NS
EPT = E // NW
C = 100
NCHUNK = EPT // C
SUP = 5
NSUP = NCHUNK // SUP
NRING = 3
RPT = N // NS
WRT = 624
BN = 2000
NB = N // BN


def _agg_body(hf, ei_hbm, agg_hbm, srcb, dstb, rows_v, agg_sh, gsem, isem):
    c = lax.axis_index("c")
    s = lax.axis_index("s")
    w = c * NS + s

    def _zr(i, carry):
        for j in range(H // 16):
            rows_v[0, i, pl.ds(j * 16, 16)] = jnp.zeros((16,), jnp.float32)
        return carry
    lax.fori_loop(0, C, _zr, 0)
    for k in range(RPT // C):
        pltpu.sync_copy(rows_v.at[0], agg_sh.at[pl.ds(s * RPT + k * C, C)])
    _tail = RPT - (RPT // C) * C
    if _tail:
        pltpu.sync_copy(rows_v.at[0, pl.ds(0, _tail)],
                        agg_sh.at[pl.ds(s * RPT + (RPT // C) * C, _tail)])
    plsc.subcore_barrier()

    pltpu.sync_copy(ei_hbm.at[0, w, 0], srcb.at[0])
    pltpu.sync_copy(ei_hbm.at[1, w, 0], dstb.at[0])
    pltpu.sync_copy(ei_hbm.at[0, w, 1], srcb.at[1])
    pltpu.sync_copy(ei_hbm.at[1, w, 1], dstb.at[1])

    def _issue(i):
        m = lax.div(i, SUP)
        j = lax.rem(i, SUP)
        pltpu.async_copy(hf.at[srcb.at[lax.rem(m, 2), j]],
                         rows_v.at[lax.rem(i, NRING)], gsem)

    _issue(0)
    _issue(1)

    def _sup(m, carry):
        @pl.when(jnp.logical_and(m >= 1, m < NSUP - 1))
        def _():
            pltpu.make_async_copy(ei_hbm.at[0, w, 0], srcb.at[0], isem).wait()
            pltpu.make_async_copy(ei_hbm.at[1, w, 0], dstb.at[0], isem).wait()

        base = m * SUP
        for j in range(SUP):
            i = base + j
            nxt = i + 2

            @pl.when(nxt < NCHUNK)
            def _():
                _issue(nxt)

            slot = lax.rem(i, NRING)
            pltpu.make_async_copy(hf.at[srcb.at[0, 0]], rows_v.at[slot],
                                  gsem).wait()
            pltpu.sync_copy(rows_v.at[slot],
                            agg_sh.at[dstb.at[lax.rem(m, 2), j]], add=True)

        @pl.when(m + 2 < NSUP)
        def _():
            pltpu.async_copy(ei_hbm.at[0, w, m + 2], srcb.at[lax.rem(m, 2)],
                             isem)
            pltpu.async_copy(ei_hbm.at[1, w, m + 2], dstb.at[lax.rem(m, 2)],
                             isem)
        return carry
    lax.fori_loop(0, NSUP, _sup, 0)
    plsc.subcore_barrier()

    @pl.when(s < NS - 1)
    def _():
        pltpu.sync_copy(agg_sh.at[pl.ds(s * WRT, WRT)],
                        agg_hbm.at[c, pl.ds(s * WRT, WRT)])

    @pl.when(s == NS - 1)
    def _():
        pltpu.sync_copy(agg_sh.at[pl.ds((NS - 1) * WRT, N - (NS - 1) * WRT)],
                        agg_hbm.at[c, pl.ds((NS - 1) * WRT,
                                            N - (NS - 1) * WRT)])


_agg = pl.kernel(
    _agg_body,
    out_type=jax.ShapeDtypeStruct((NC, N, H), jnp.float32),
    mesh=plsc.VectorSubcoreMesh(core_axis_name="c", subcore_axis_name="s",
                                num_cores=NC, num_subcores=NS),
    scratch_types=[
        pltpu.VMEM((2, SUP, C), jnp.int32),
        pltpu.VMEM((2, SUP, C), jnp.int32),
        pltpu.VMEM((NRING, C, H), jnp.float32),
        pltpu.VMEM_SHARED((N, H), jnp.float32),
        pltpu.SemaphoreType.DMA,
        pltpu.SemaphoreType.DMA,
    ],
)


def _enc_body(x_ref, we_ref, be_ref, h_ref):
    h_ref[...] = jnp.dot(x_ref[...], we_ref[...],
                         preferred_element_type=jnp.float32) + be_ref[...]


_enc = pl.pallas_call(
    _enc_body,
    grid=(NB,),
    in_specs=[
        pl.BlockSpec((BN, H), lambda i: (i, 0)),
        pl.BlockSpec((H, H), lambda i: (0, 0)),
        pl.BlockSpec((1, H), lambda i: (0, 0)),
    ],
    out_specs=pl.BlockSpec((BN, H), lambda i: (i, 0)),
    out_shape=jax.ShapeDtypeStruct((N, H), jnp.float32),
)


def _self_body(h_ref, ws_ref, b_ref, s_ref):
    s_ref[...] = jnp.dot(h_ref[...], ws_ref[...],
                         preferred_element_type=jnp.float32) + b_ref[...]


_self = pl.pallas_call(
    _self_body,
    grid=(NB,),
    in_specs=[
        pl.BlockSpec((BN, H), lambda i: (i, 0)),
        pl.BlockSpec((H, H), lambda i: (0, 0)),
        pl.BlockSpec((1, H), lambda i: (0, 0)),
    ],
    out_specs=pl.BlockSpec((BN, H), lambda i: (i, 0)),
    out_shape=jax.ShapeDtypeStruct((N, H), jnp.float32),
)


def _combine_body(self_ref, a_ref, wn_ref, h_ref):
    agg = a_ref[0] + a_ref[1]
    h_ref[...] = jnp.maximum(
        self_ref[...] + jnp.dot(agg, wn_ref[...],
                                preferred_element_type=jnp.float32), 0.0)


_combine = pl.pallas_call(
    _combine_body,
    grid=(NB,),
    in_specs=[
        pl.BlockSpec((BN, H), lambda i: (i, 0)),
        pl.BlockSpec((NC, BN, H), lambda i: (0, i, 0)),
        pl.BlockSpec((H, H), lambda i: (0, 0)),
    ],
    out_specs=pl.BlockSpec((BN, H), lambda i: (i, 0)),
    out_shape=jax.ShapeDtypeStruct((N, H), jnp.float32),
)


def _last_body(self_ref, a_ref, wn_ref, batch_ref, wd_ref, bd_ref, o_ref):
    i = pl.program_id(0)
    agg = a_ref[0] + a_ref[1]
    h3 = self_ref[...] + jnp.dot(agg, wn_ref[...],
                                 preferred_element_type=jnp.float32)
    bvec = batch_ref[0]
    rows = lax.broadcasted_iota(jnp.int32, (G, BN), 0)
    m = (rows == bvec).astype(jnp.float32)
    g = jnp.dot(m, h3, preferred_element_type=jnp.float32)
    contrib = jnp.dot(g, wd_ref[...], preferred_element_type=jnp.float32)

    @pl.when(i == 0)
    def _():
        o_ref[...] = contrib + bd_ref[...]

    @pl.when(i > 0)
    def _():
        o_ref[...] += contrib


_last = pl.pallas_call(
    _last_body,
    grid=(NB,),
    in_specs=[
        pl.BlockSpec((BN, H), lambda i: (i, 0)),
        pl.BlockSpec((NC, BN, H), lambda i: (0, i, 0)),
        pl.BlockSpec((H, H), lambda i: (0, 0)),
        pl.BlockSpec((1, 1, BN), lambda i: (i, 0, 0)),
        pl.BlockSpec((H, OUT), lambda i: (0, 0)),
        pl.BlockSpec((1, OUT), lambda i: (0, 0)),
    ],
    out_specs=pl.BlockSpec((G, OUT), lambda i: (0, 0)),
    out_shape=jax.ShapeDtypeStruct((G, OUT), jnp.float32),
)


def kernel(x, edge_attr, edge_index, batch, W_enc, b_enc, W_self0, W_nbr0, b0,
           W_self1, W_nbr1, b1, W_self2, W_nbr2, b2, W_dec, b_dec):
    ei5 = edge_index.reshape(2, NW, NSUP, SUP, C)
    batch3 = batch.reshape(NB, 1, BN)

    h = _enc(x, W_enc, b_enc.reshape(1, H))
    a0 = _agg(h, ei5)
    self0 = _self(h, W_self0, b0.reshape(1, H))
    h = _combine(self0, a0, W_nbr0)
    a1 = _agg(h, ei5)
    self1 = _self(h, W_self1, b1.reshape(1, H))
    h = _combine(self1, a1, W_nbr1)
    a2 = _agg(h, ei5)
    self2 = _self(h, W_self2, b2.reshape(1, H))
    return _last(self2, a2, W_nbr2, batch3, W_dec, b_dec.reshape(1, OUT))
